# baseline (speedup 1.0000x reference)
import functools

import jax
import jax.numpy as jnp
from jax import lax
from jax.experimental import pallas as pl
from jax.experimental.pallas import tpu as pltpu
from jax.experimental.pallas import tpu_sc as plsc

VOCAB_N = 100000
EMBED_D = 512
SEQ_N = 50
BATCH_N = 1024

NC = 2
NS = 16
NW = NC * NS

ROWS_TOTAL = BATCH_N * SEQ_N
ROWS_PER_W = ROWS_TOTAL // NW
CHUNK = 2 * SEQ_N
CHUNKS_PER_W = ROWS_PER_W // CHUNK


def _pe_table():
    i = jnp.arange(SEQ_N, dtype=jnp.float32)[:, None]
    j = jnp.arange(EMBED_D // 2, dtype=jnp.float32)[None, :]
    ang = i / jnp.power(jnp.float32(10000.0), 2.0 * j / EMBED_D)
    return jnp.stack([jnp.sin(ang), jnp.cos(ang)], axis=-1).reshape(SEQ_N, EMBED_D)


def _body(table_hbm, idx_hbm, pe_hbm, out_hbm,
          idx_v, buf_v, pe_v, gsem, wsem):
    wid = lax.axis_index("s") * NC + lax.axis_index("c")
    j0 = wid * CHUNKS_PER_W

    pltpu.sync_copy(pe_hbm, pe_v)

    for k in range(CHUNKS_PER_W):
        j = j0 + k
        pltpu.sync_copy(idx_hbm.at[j], idx_v)
        pltpu.async_copy(table_hbm.at[idx_v], buf_v, gsem).wait()

        def add_pe(s, _):
            for v in range(0, EMBED_D, 16):
                pev = pe_v[s, pl.ds(v, 16)]
                plsc.addupdate(buf_v.at[s, pl.ds(v, 16)], pev)
                plsc.addupdate(buf_v.at[s + SEQ_N, pl.ds(v, 16)], pev)
            return 0

        lax.fori_loop(0, SEQ_N, add_pe, 0)
        pltpu.async_copy(buf_v, out_hbm.at[j], wsem).wait()


@jax.jit
def _run(x, table, pe):
    idx = x.reshape(ROWS_TOTAL // CHUNK, CHUNK)
    mesh = plsc.VectorSubcoreMesh(core_axis_name="c", subcore_axis_name="s")
    out = pl.kernel(
        _body,
        out_type=jax.ShapeDtypeStruct(
            (ROWS_TOTAL // CHUNK, CHUNK, EMBED_D), jnp.float32),
        mesh=mesh,
        scratch_types=[
            pltpu.VMEM((CHUNK,), jnp.int32),
            pltpu.VMEM((CHUNK, EMBED_D), jnp.float32),
            pltpu.VMEM((SEQ_N, EMBED_D), jnp.float32),
            pltpu.SemaphoreType.DMA,
            pltpu.SemaphoreType.DMA,
        ],
    )(table, idx, pe)
    return out.reshape(BATCH_N, SEQ_N, EMBED_D)


def kernel(x, offsets, table):
    del offsets
    return _run(x, table, _pe_table())
